# baseline (speedup 1.0000x reference)
import functools

import jax
import jax.numpy as jnp
from jax import lax
from jax.experimental import pallas as pl
from jax.experimental.pallas import tpu as pltpu
from jax.experimental.pallas import tpu_sc as plsc

N = 10000
E = 320000
D = 128
BN = 1000
BE = 8000
CHUNKS = (64000, 128000, 128000)
NC = 2
NS = 16
TILES = NC * NS
GW = 80
SW = 40
R = 5
NP = 10240
SROWS = NP // NS
EPS = 1e-5

_HIGH = jax.lax.Precision.HIGHEST


def _gn(x, w, b):
    mu = jnp.mean(x, axis=1, keepdims=True)
    var = jnp.mean((x - mu) * (x - mu), axis=1, keepdims=True)
    return (x - mu) * jax.lax.rsqrt(var + EPS) * w + b


def _dot(x, w):
    return jax.lax.dot_general(x, w, (((1,), (0,)), ((), ())),
                               precision=_HIGH, preferred_element_type=jnp.float32)


def _dot_bf16(x, w):
    return jax.lax.dot_general(x.astype(jnp.bfloat16), w.astype(jnp.bfloat16),
                               (((1,), (0,)), ((), ())),
                               preferred_element_type=jnp.float32)


def _pack2(feat, cproj):
    fu = jax.lax.bitcast_convert_type(
        feat.astype(jnp.bfloat16).astype(jnp.float32), jnp.uint32)
    cu = jax.lax.bitcast_convert_type(
        cproj.astype(jnp.bfloat16).astype(jnp.float32), jnp.uint32)
    packed = (fu >> 16) | (cu & jnp.uint32(0xFFFF0000))
    return jax.lax.bitcast_convert_type(packed, jnp.float32)


def _unpack2(packed):
    u = jax.lax.bitcast_convert_type(packed, jnp.uint32)
    feat = jax.lax.bitcast_convert_type(u << 16, jnp.float32)
    cproj = jax.lax.bitcast_convert_type(u & jnp.uint32(0xFFFF0000), jnp.float32)
    return feat, cproj


def _prologue_body(agts_ref, ctrp_ref, dw0_ref, q_w_ref, q_gw_ref, q_gb_ref,
                   ctx_w0_ref, agt_w_ref, th_ref, tw_ref, ag_ref):
    x = agts_ref[...]
    qn = jax.nn.relu(_gn(_dot(x, q_w_ref[...]), q_gw_ref[...], q_gb_ref[...]))
    a = _dot(qn, ctx_w0_ref[D:2 * D, :])
    b = _dot(x, ctx_w0_ref[2 * D:, :])
    ctrp = ctrp_ref[...]
    c = (ctrp[:, 0:1] * dw0_ref[0:1, :] + ctrp[:, 1:2] * dw0_ref[1:2, :])
    th_ref[...] = _pack2(a, c)
    tw_ref[...] = _pack2(b, c)
    ag_ref[...] = _dot(x, agt_w_ref[...])


def _prologue(agts, ctrp, dist_w0, q_w, q_gw, q_gb, ctx_w0, agt_w):
    grid = (N // BN,)
    nd = jax.ShapeDtypeStruct((N, D), jnp.float32)
    return pl.pallas_call(
        _prologue_body,
        grid=grid,
        in_specs=[
            pl.BlockSpec((BN, D), lambda i: (i, 0)),
            pl.BlockSpec((BN, 128), lambda i: (i, 0)),
            pl.BlockSpec((2, D), lambda i: (0, 0)),
            pl.BlockSpec((D, D), lambda i: (0, 0)),
            pl.BlockSpec((1, D), lambda i: (0, 0)),
            pl.BlockSpec((1, D), lambda i: (0, 0)),
            pl.BlockSpec((3 * D, D), lambda i: (0, 0)),
            pl.BlockSpec((D, D), lambda i: (0, 0)),
        ],
        out_specs=[pl.BlockSpec((BN, D), lambda i: (i, 0))] * 3,
        out_shape=[nd, nd, nd],
    )(agts, ctrp, dist_w0, q_w, q_gw, q_gb, ctx_w0, agt_w)


def _sc_mesh():
    return plsc.VectorSubcoreMesh(core_axis_name="c", subcore_axis_name="s",
                                  num_cores=NC, num_subcores=NS)


def _sc_gather(th_t, tw_t, hi, wi, ec):
    ept = ec // TILES
    gng = ept // (GW * R)
    ed = jax.ShapeDtypeStruct((ec, D), jnp.float32)

    @functools.partial(
        pl.kernel,
        out_type=(ed, ed),
        mesh=_sc_mesh(),
        scratch_types=(
            [pltpu.VMEM((ept,), jnp.int32)] * 2
            + [pltpu.VMEM((GW, D), jnp.float32)] * (2 * R)
            + [pltpu.SemaphoreType.DMA] * (2 * R)
        ),
    )
    def k(th_hbm, tw_hbm, hi_hbm, wi_hbm, oh_hbm, ow_hbm, *scr):
        ihbuf, iwbuf = scr[0], scr[1]
        bufs = [scr[2 + 2 * r: 2 + 2 * (r + 1)] for r in range(R)]
        semg = scr[2 + 2 * R: 2 + 3 * R]
        semw = scr[2 + 3 * R: 2 + 4 * R]
        tile = lax.axis_index("c") * NS + lax.axis_index("s")
        base = tile * ept
        pltpu.sync_copy(hi_hbm.at[pl.ds(base, ept)], ihbuf)
        pltpu.sync_copy(wi_hbm.at[pl.ds(base, ept)], iwbuf)

        outs = (oh_hbm, ow_hbm)

        def wb_descs(r, w0):
            off = base + w0 * GW
            return [pltpu.make_async_copy(bufs[r][t], outs[t].at[pl.ds(off, GW)],
                                          semw[r]) for t in range(2)]

        @pl.loop(0, gng)
        def _(st):
            w0 = st * R
            for r in range(R):
                @pl.when(st > 0)
                def _():
                    for dsc in wb_descs(r, w0 + r):
                        dsc.wait()
                ih = ihbuf.at[pl.ds((w0 + r) * GW, GW)]
                iw = iwbuf.at[pl.ds((w0 + r) * GW, GW)]
                pltpu.async_copy(th_hbm.at[ih], bufs[r][0], semg[r])
                pltpu.async_copy(tw_hbm.at[iw], bufs[r][1], semg[r])
            for r in range(R):
                ih = ihbuf.at[pl.ds((w0 + r) * GW, GW)]
                pltpu.make_async_copy(th_hbm.at[ih], bufs[r][0], semg[r]).wait()
                pltpu.make_async_copy(th_hbm.at[ih], bufs[r][1], semg[r]).wait()
                off = base + (w0 + r) * GW
                for t in range(2):
                    pltpu.async_copy(bufs[r][t], outs[t].at[pl.ds(off, GW)],
                                     semw[r])

        for r in range(R):
            for dsc in wb_descs(r, (gng - 1) * R + r):
                dsc.wait()

    return k(th_t, tw_t, hi, wi)


def _edge_body(gh_ref, gw_ref, dw0b_ref, dw1_ref, dgw_ref,
               dgb_ref, ctx_w0_ref, cgw_ref, cgb_ref, g_ref):
    ah, ch = _unpack2(gh_ref[...])
    bw, cw = _unpack2(gw_ref[...])
    d1 = jax.nn.relu(ch.astype(jnp.float32) - cw.astype(jnp.float32)
                     + dw0b_ref[...])
    d2 = jax.nn.relu(_gn(_dot_bf16(d1, dw1_ref[...]), dgw_ref[...], dgb_ref[...]))
    mp = (_dot_bf16(d2, ctx_w0_ref[0:D, :]) + ah.astype(jnp.float32)
          + bw.astype(jnp.float32))
    g_ref[...] = jax.nn.relu(_gn(mp, cgw_ref[...], cgb_ref[...]))


def _edge_mlp(gh, gw, dist_b0, dist_w1, dist_gw, dist_gb,
              ctx_w0, ctx_gw, ctx_gb, ec):
    grid = (ec // BE,)
    return pl.pallas_call(
        _edge_body,
        grid=grid,
        in_specs=[
            pl.BlockSpec((BE, D), lambda i: (i, 0)),
            pl.BlockSpec((BE, D), lambda i: (i, 0)),
            pl.BlockSpec((1, D), lambda i: (0, 0)),
            pl.BlockSpec((D, D), lambda i: (0, 0)),
            pl.BlockSpec((1, D), lambda i: (0, 0)),
            pl.BlockSpec((1, D), lambda i: (0, 0)),
            pl.BlockSpec((3 * D, D), lambda i: (0, 0)),
            pl.BlockSpec((1, D), lambda i: (0, 0)),
            pl.BlockSpec((1, D), lambda i: (0, 0)),
        ],
        out_specs=pl.BlockSpec((BE, D), lambda i: (i, 0)),
        out_shape=jax.ShapeDtypeStruct((ec, D), jnp.float32),
    )(gh, gw, dist_b0, dist_w1, dist_gw, dist_gb,
      ctx_w0, ctx_gw, ctx_gb)


def _sc_scatter(g, hi, zeros640, ec):
    ept = ec // TILES
    sng = ept // (SW * R)

    @functools.partial(
        pl.kernel,
        out_type=jax.ShapeDtypeStruct((NC, NP, D), jnp.float32),
        mesh=_sc_mesh(),
        scratch_types=(
            [pltpu.VMEM_SHARED((NP, D), jnp.float32)]
            + [pltpu.VMEM((SW, D), jnp.float32)] * R
            + [pltpu.VMEM((SW,), jnp.int32)] * R
            + [pltpu.SemaphoreType.DMA] * (2 * R)
        ),
    )
    def k(g_hbm, hi_hbm, z_hbm, s_hbm, *scr):
        shared = scr[0]
        gbufs = scr[1: 1 + R]
        ibufs = scr[1 + R: 1 + 2 * R]
        seml = scr[1 + 2 * R: 1 + 3 * R]
        sems = scr[1 + 3 * R: 1 + 4 * R]
        c = lax.axis_index("c")
        s = lax.axis_index("s")
        base = (c * NS + s) * ept
        pltpu.sync_copy(z_hbm, shared.at[pl.ds(s * SROWS, SROWS)])
        plsc.subcore_barrier()

        @pl.loop(0, sng)
        def _(st):
            w0 = st * R
            for r in range(R):
                @pl.when(st > 0)
                def _():
                    pltpu.make_async_copy(gbufs[r], shared.at[ibufs[r]],
                                          sems[r]).wait()
                off = base + (w0 + r) * SW
                pltpu.async_copy(g_hbm.at[pl.ds(off, SW)], gbufs[r], seml[r])
                pltpu.async_copy(hi_hbm.at[pl.ds(off, SW)], ibufs[r], seml[r])
            for r in range(R):
                off = base + (w0 + r) * SW
                pltpu.make_async_copy(g_hbm.at[pl.ds(off, SW)], gbufs[r],
                                      seml[r]).wait()
                pltpu.make_async_copy(hi_hbm.at[pl.ds(off, SW)], ibufs[r],
                                      seml[r]).wait()
                pltpu.async_copy(gbufs[r], shared.at[ibufs[r]], sems[r],
                                 add=True)

        for r in range(R):
            pltpu.make_async_copy(gbufs[r], shared.at[ibufs[r]], sems[r]).wait()
        plsc.subcore_barrier()
        pltpu.sync_copy(shared.at[pl.ds(s * SROWS, SROWS)],
                        s_hbm.at[c].at[pl.ds(s * SROWS, SROWS)])

    return k(g, hi, zeros640)


def _epilogue_body(*refs):
    s_refs = refs[:2 * len(CHUNKS)]
    (ag_ref, agts_ref, ctx_w1_ref, lin_w_ref,
     nw_ref, nb_ref, lgw_ref, lgb_ref, out_ref) = refs[2 * len(CHUNKS):]
    acc = s_refs[0][...]
    for sr in s_refs[1:]:
        acc = acc + sr[...]
    h = _dot(acc, ctx_w1_ref[...]) + ag_ref[...]
    h = jax.nn.relu(_gn(h, nw_ref[...], nb_ref[...]))
    o = _gn(_dot(h, lin_w_ref[...]), lgw_ref[...], lgb_ref[...])
    out_ref[...] = jax.nn.relu(o + agts_ref[...])


def _epilogue(parts, ag, agts, ctx_w1, lin_w, norm_w, norm_b, lin_gw, lin_gb):
    grid = (N // BN,)
    blk = pl.BlockSpec((BN, D), lambda i: (i, 0))
    full = pl.BlockSpec((D, D), lambda i: (0, 0))
    row = pl.BlockSpec((1, D), lambda i: (0, 0))
    return pl.pallas_call(
        _epilogue_body,
        grid=grid,
        in_specs=[blk] * len(parts) + [blk, blk, full, full, row, row, row, row],
        out_specs=blk,
        out_shape=jax.ShapeDtypeStruct((N, D), jnp.float32),
    )(*parts, ag, agts, ctx_w1, lin_w, norm_w, norm_b, lin_gw, lin_gb)


def kernel(agts, agt_ctrs, edge_index, dist_w0, dist_b0, dist_w1, dist_gw,
           dist_gb, q_w, q_gw, q_gb, ctx_w0, ctx_gw, ctx_gb, ctx_w1, agt_w,
           norm_w, norm_b, lin_w, lin_gw, lin_gb):
    hi = edge_index[0]
    wi = edge_index[1]
    ctrp = jnp.pad(agt_ctrs, ((0, 0), (0, 128 - 2)))
    r1 = lambda v: v.reshape(1, D)

    th_t, tw_t, ag = _prologue(agts, ctrp, dist_w0, q_w, r1(q_gw), r1(q_gb),
                               ctx_w0, agt_w)
    zeros640 = jnp.zeros((SROWS, D), jnp.float32)

    parts = []
    off = 0
    for ec in CHUNKS:
        hic = lax.slice(hi, (off,), (off + ec,))
        wic = lax.slice(wi, (off,), (off + ec,))
        gh, gw = _sc_gather(th_t, tw_t, hic, wic, ec)
        g = _edge_mlp(gh, gw, r1(dist_b0), dist_w1, r1(dist_gw),
                      r1(dist_gb), ctx_w0, r1(ctx_gw), r1(ctx_gb), ec)
        s2 = _sc_scatter(g, hic, zeros640, ec)
        parts.extend([s2[0, :N], s2[1, :N]])
        off += ec

    out = _epilogue(parts, ag, agts, ctx_w1, lin_w, r1(norm_w),
                    r1(norm_b), r1(lin_gw), r1(lin_gb))
    return out


# speedup vs baseline: 1.0322x; 1.0322x over previous
import functools

import jax
import jax.numpy as jnp
from jax import lax
from jax.experimental import pallas as pl
from jax.experimental.pallas import tpu as pltpu
from jax.experimental.pallas import tpu_sc as plsc

N = 10000
E = 320000
D = 128
BN = 1000
BE = 8000
CHUNKS = (192000, 128000)
NC = 2
NS = 16
TILES = NC * NS
GW = 80
SW = 40
R = 5
NP = 10240
SROWS = NP // NS
EPS = 1e-5

_HIGH = jax.lax.Precision.HIGHEST


def _gn(x, w, b):
    mu = jnp.mean(x, axis=1, keepdims=True)
    var = jnp.mean((x - mu) * (x - mu), axis=1, keepdims=True)
    return (x - mu) * jax.lax.rsqrt(var + EPS) * w + b


def _dot(x, w):
    return jax.lax.dot_general(x, w, (((1,), (0,)), ((), ())),
                               precision=_HIGH, preferred_element_type=jnp.float32)


def _dot_bf16(x, w):
    return jax.lax.dot_general(x.astype(jnp.bfloat16), w.astype(jnp.bfloat16),
                               (((1,), (0,)), ((), ())),
                               preferred_element_type=jnp.float32)


def _pack2(feat, cproj):
    fu = jax.lax.bitcast_convert_type(
        feat.astype(jnp.bfloat16).astype(jnp.float32), jnp.uint32)
    cu = jax.lax.bitcast_convert_type(
        cproj.astype(jnp.bfloat16).astype(jnp.float32), jnp.uint32)
    packed = (fu >> 16) | (cu & jnp.uint32(0xFFFF0000))
    return jax.lax.bitcast_convert_type(packed, jnp.float32)


def _unpack2(packed):
    u = jax.lax.bitcast_convert_type(packed, jnp.uint32)
    feat = jax.lax.bitcast_convert_type(u << 16, jnp.float32)
    cproj = jax.lax.bitcast_convert_type(u & jnp.uint32(0xFFFF0000), jnp.float32)
    return feat, cproj


def _prologue_body(agts_ref, ctrp_ref, dw0_ref, q_w_ref, q_gw_ref, q_gb_ref,
                   ctx_w0_ref, agt_w_ref, th_ref, tw_ref, ag_ref):
    x = agts_ref[...]
    qn = jax.nn.relu(_gn(_dot(x, q_w_ref[...]), q_gw_ref[...], q_gb_ref[...]))
    a = _dot(qn, ctx_w0_ref[D:2 * D, :])
    b = _dot(x, ctx_w0_ref[2 * D:, :])
    ctrp = ctrp_ref[...]
    c = (ctrp[:, 0:1] * dw0_ref[0:1, :] + ctrp[:, 1:2] * dw0_ref[1:2, :])
    th_ref[...] = _pack2(a, c)
    tw_ref[...] = _pack2(b, c)
    ag_ref[...] = _dot(x, agt_w_ref[...])


def _prologue(agts, ctrp, dist_w0, q_w, q_gw, q_gb, ctx_w0, agt_w):
    grid = (N // BN,)
    nd = jax.ShapeDtypeStruct((N, D), jnp.float32)
    return pl.pallas_call(
        _prologue_body,
        grid=grid,
        in_specs=[
            pl.BlockSpec((BN, D), lambda i: (i, 0)),
            pl.BlockSpec((BN, 128), lambda i: (i, 0)),
            pl.BlockSpec((2, D), lambda i: (0, 0)),
            pl.BlockSpec((D, D), lambda i: (0, 0)),
            pl.BlockSpec((1, D), lambda i: (0, 0)),
            pl.BlockSpec((1, D), lambda i: (0, 0)),
            pl.BlockSpec((3 * D, D), lambda i: (0, 0)),
            pl.BlockSpec((D, D), lambda i: (0, 0)),
        ],
        out_specs=[pl.BlockSpec((BN, D), lambda i: (i, 0))] * 3,
        out_shape=[nd, nd, nd],
    )(agts, ctrp, dist_w0, q_w, q_gw, q_gb, ctx_w0, agt_w)


def _sc_mesh():
    return plsc.VectorSubcoreMesh(core_axis_name="c", subcore_axis_name="s",
                                  num_cores=NC, num_subcores=NS)


def _sc_gather(th_t, tw_t, hi, wi, ec):
    ept = ec // TILES
    gng = ept // (GW * R)
    ed = jax.ShapeDtypeStruct((ec, D), jnp.float32)

    @functools.partial(
        pl.kernel,
        out_type=(ed, ed),
        mesh=_sc_mesh(),
        scratch_types=(
            [pltpu.VMEM((ept,), jnp.int32)] * 2
            + [pltpu.VMEM((GW, D), jnp.float32)] * (2 * R)
            + [pltpu.SemaphoreType.DMA] * (2 * R)
        ),
    )
    def k(th_hbm, tw_hbm, hi_hbm, wi_hbm, oh_hbm, ow_hbm, *scr):
        ihbuf, iwbuf = scr[0], scr[1]
        bufs = [scr[2 + 2 * r: 2 + 2 * (r + 1)] for r in range(R)]
        semg = scr[2 + 2 * R: 2 + 3 * R]
        semw = scr[2 + 3 * R: 2 + 4 * R]
        tile = lax.axis_index("c") * NS + lax.axis_index("s")
        base = tile * ept
        pltpu.sync_copy(hi_hbm.at[pl.ds(base, ept)], ihbuf)
        pltpu.sync_copy(wi_hbm.at[pl.ds(base, ept)], iwbuf)

        outs = (oh_hbm, ow_hbm)

        def wb_descs(r, w0):
            off = base + w0 * GW
            return [pltpu.make_async_copy(bufs[r][t], outs[t].at[pl.ds(off, GW)],
                                          semw[r]) for t in range(2)]

        @pl.loop(0, gng)
        def _(st):
            w0 = st * R
            for r in range(R):
                @pl.when(st > 0)
                def _():
                    for dsc in wb_descs(r, w0 + r):
                        dsc.wait()
                ih = ihbuf.at[pl.ds((w0 + r) * GW, GW)]
                iw = iwbuf.at[pl.ds((w0 + r) * GW, GW)]
                pltpu.async_copy(th_hbm.at[ih], bufs[r][0], semg[r])
                pltpu.async_copy(tw_hbm.at[iw], bufs[r][1], semg[r])
            for r in range(R):
                ih = ihbuf.at[pl.ds((w0 + r) * GW, GW)]
                pltpu.make_async_copy(th_hbm.at[ih], bufs[r][0], semg[r]).wait()
                pltpu.make_async_copy(th_hbm.at[ih], bufs[r][1], semg[r]).wait()
                off = base + (w0 + r) * GW
                for t in range(2):
                    pltpu.async_copy(bufs[r][t], outs[t].at[pl.ds(off, GW)],
                                     semw[r])

        for r in range(R):
            for dsc in wb_descs(r, (gng - 1) * R + r):
                dsc.wait()

    return k(th_t, tw_t, hi, wi)


def _edge_body(gh_ref, gw_ref, dw0b_ref, dw1_ref, dgw_ref,
               dgb_ref, ctx_w0_ref, cgw_ref, cgb_ref, g_ref):
    ah, ch = _unpack2(gh_ref[...])
    bw, cw = _unpack2(gw_ref[...])
    d1 = jax.nn.relu(ch.astype(jnp.float32) - cw.astype(jnp.float32)
                     + dw0b_ref[...])
    d2 = jax.nn.relu(_gn(_dot_bf16(d1, dw1_ref[...]), dgw_ref[...], dgb_ref[...]))
    mp = (_dot_bf16(d2, ctx_w0_ref[0:D, :]) + ah.astype(jnp.float32)
          + bw.astype(jnp.float32))
    g_ref[...] = jax.nn.relu(_gn(mp, cgw_ref[...], cgb_ref[...]))


def _edge_mlp(gh, gw, dist_b0, dist_w1, dist_gw, dist_gb,
              ctx_w0, ctx_gw, ctx_gb, ec):
    grid = (ec // BE,)
    return pl.pallas_call(
        _edge_body,
        grid=grid,
        in_specs=[
            pl.BlockSpec((BE, D), lambda i: (i, 0)),
            pl.BlockSpec((BE, D), lambda i: (i, 0)),
            pl.BlockSpec((1, D), lambda i: (0, 0)),
            pl.BlockSpec((D, D), lambda i: (0, 0)),
            pl.BlockSpec((1, D), lambda i: (0, 0)),
            pl.BlockSpec((1, D), lambda i: (0, 0)),
            pl.BlockSpec((3 * D, D), lambda i: (0, 0)),
            pl.BlockSpec((1, D), lambda i: (0, 0)),
            pl.BlockSpec((1, D), lambda i: (0, 0)),
        ],
        out_specs=pl.BlockSpec((BE, D), lambda i: (i, 0)),
        out_shape=jax.ShapeDtypeStruct((ec, D), jnp.float32),
    )(gh, gw, dist_b0, dist_w1, dist_gw, dist_gb,
      ctx_w0, ctx_gw, ctx_gb)


def _sc_scatter(g, hi, zeros640, ec):
    ept = ec // TILES
    sng = ept // (SW * R)

    @functools.partial(
        pl.kernel,
        out_type=jax.ShapeDtypeStruct((NC, NP, D), jnp.float32),
        mesh=_sc_mesh(),
        scratch_types=(
            [pltpu.VMEM_SHARED((NP, D), jnp.float32)]
            + [pltpu.VMEM((SW, D), jnp.float32)] * R
            + [pltpu.VMEM((SW,), jnp.int32)] * R
            + [pltpu.SemaphoreType.DMA] * (2 * R)
        ),
    )
    def k(g_hbm, hi_hbm, z_hbm, s_hbm, *scr):
        shared = scr[0]
        gbufs = scr[1: 1 + R]
        ibufs = scr[1 + R: 1 + 2 * R]
        seml = scr[1 + 2 * R: 1 + 3 * R]
        sems = scr[1 + 3 * R: 1 + 4 * R]
        c = lax.axis_index("c")
        s = lax.axis_index("s")
        base = (c * NS + s) * ept
        pltpu.sync_copy(z_hbm, shared.at[pl.ds(s * SROWS, SROWS)])
        plsc.subcore_barrier()

        @pl.loop(0, sng)
        def _(st):
            w0 = st * R
            for r in range(R):
                @pl.when(st > 0)
                def _():
                    pltpu.make_async_copy(gbufs[r], shared.at[ibufs[r]],
                                          sems[r]).wait()
                off = base + (w0 + r) * SW
                pltpu.async_copy(g_hbm.at[pl.ds(off, SW)], gbufs[r], seml[r])
                pltpu.async_copy(hi_hbm.at[pl.ds(off, SW)], ibufs[r], seml[r])
            for r in range(R):
                off = base + (w0 + r) * SW
                pltpu.make_async_copy(g_hbm.at[pl.ds(off, SW)], gbufs[r],
                                      seml[r]).wait()
                pltpu.make_async_copy(hi_hbm.at[pl.ds(off, SW)], ibufs[r],
                                      seml[r]).wait()
                pltpu.async_copy(gbufs[r], shared.at[ibufs[r]], sems[r],
                                 add=True)

        for r in range(R):
            pltpu.make_async_copy(gbufs[r], shared.at[ibufs[r]], sems[r]).wait()
        plsc.subcore_barrier()
        pltpu.sync_copy(shared.at[pl.ds(s * SROWS, SROWS)],
                        s_hbm.at[c].at[pl.ds(s * SROWS, SROWS)])

    return k(g, hi, zeros640)


def _epilogue_body(*refs):
    s_refs = refs[:2 * len(CHUNKS)]
    (ag_ref, agts_ref, ctx_w1_ref, lin_w_ref,
     nw_ref, nb_ref, lgw_ref, lgb_ref, out_ref) = refs[2 * len(CHUNKS):]
    acc = s_refs[0][...]
    for sr in s_refs[1:]:
        acc = acc + sr[...]
    h = _dot(acc, ctx_w1_ref[...]) + ag_ref[...]
    h = jax.nn.relu(_gn(h, nw_ref[...], nb_ref[...]))
    o = _gn(_dot(h, lin_w_ref[...]), lgw_ref[...], lgb_ref[...])
    out_ref[...] = jax.nn.relu(o + agts_ref[...])


def _epilogue(parts, ag, agts, ctx_w1, lin_w, norm_w, norm_b, lin_gw, lin_gb):
    grid = (N // BN,)
    blk = pl.BlockSpec((BN, D), lambda i: (i, 0))
    full = pl.BlockSpec((D, D), lambda i: (0, 0))
    row = pl.BlockSpec((1, D), lambda i: (0, 0))
    return pl.pallas_call(
        _epilogue_body,
        grid=grid,
        in_specs=[blk] * len(parts) + [blk, blk, full, full, row, row, row, row],
        out_specs=blk,
        out_shape=jax.ShapeDtypeStruct((N, D), jnp.float32),
    )(*parts, ag, agts, ctx_w1, lin_w, norm_w, norm_b, lin_gw, lin_gb)


def kernel(agts, agt_ctrs, edge_index, dist_w0, dist_b0, dist_w1, dist_gw,
           dist_gb, q_w, q_gw, q_gb, ctx_w0, ctx_gw, ctx_gb, ctx_w1, agt_w,
           norm_w, norm_b, lin_w, lin_gw, lin_gb):
    hi = edge_index[0]
    wi = edge_index[1]
    ctrp = jnp.pad(agt_ctrs, ((0, 0), (0, 128 - 2)))
    r1 = lambda v: v.reshape(1, D)

    th_t, tw_t, ag = _prologue(agts, ctrp, dist_w0, q_w, r1(q_gw), r1(q_gb),
                               ctx_w0, agt_w)
    zeros640 = jnp.zeros((SROWS, D), jnp.float32)

    parts = []
    off = 0
    for ec in CHUNKS:
        hic = lax.slice(hi, (off,), (off + ec,))
        wic = lax.slice(wi, (off,), (off + ec,))
        gh, gw = _sc_gather(th_t, tw_t, hic, wic, ec)
        g = _edge_mlp(gh, gw, r1(dist_b0), dist_w1, r1(dist_gw),
                      r1(dist_gb), ctx_w0, r1(ctx_gw), r1(ctx_gb), ec)
        s2 = _sc_scatter(g, hic, zeros640, ec)
        parts.extend([s2[0, :N], s2[1, :N]])
        off += ec

    out = _epilogue(parts, ag, agts, ctx_w1, lin_w, r1(norm_w),
                    r1(norm_b), r1(lin_gw), r1(lin_gb))
    return out
